# baseline (speedup 1.0000x reference)
import functools

import jax
import jax.numpy as jnp
from jax import lax
from jax.experimental import pallas as pl
from jax.experimental.pallas import tpu as pltpu
from jax.experimental.pallas import tpu_sc as plsc

B = 16384
F = 26
V = 100000
D = 16
DD = 13
H0 = 128
H1 = 64

NW = 32
PER_W = B * F // NW
SUP = 8
SUP_IDX = PER_W // SUP
NROW = SUP_IDX // 128
IDX_ROWS = B * F // 128
ROWS_PER_W = IDX_ROWS // NW

BS = 512
FD = F * D


def _sc_gather(idx2d, emb_table, lin1d):
    mesh = plsc.VectorSubcoreMesh(core_axis_name="c", subcore_axis_name="s")

    @functools.partial(
        pl.kernel,
        mesh=mesh,
        compiler_params=pltpu.CompilerParams(use_tc_tiling_on_sc=False),
        out_type=[
            jax.ShapeDtypeStruct((B * F, D), jnp.float32),
            jax.ShapeDtypeStruct((B * F,), jnp.float32),
        ],
        scratch_types=[
            pltpu.VMEM((ROWS_PER_W, 128), jnp.int32),
            pltpu.VMEM((SUP_IDX, D), jnp.float32),
            pltpu.VMEM((SUP_IDX,), jnp.float32),
            pltpu.SemaphoreType.DMA,
            pltpu.SemaphoreType.DMA,
        ],
    )
    def k(idx_hbm, emb_hbm, lin_hbm, emb_out, lin_out, idx_v,
          rows_v, lin_v, sem_e, sem_l):
        wid = lax.axis_index("s") * 2 + lax.axis_index("c")
        pltpu.sync_copy(idx_hbm.at[pl.ds(wid * ROWS_PER_W, ROWS_PER_W)], idx_v)

        def body(c, _):
            base = wid * PER_W + c * SUP_IDX
            handles = []
            for j in range(NROW):
                row = c * NROW + j
                handles.append(pltpu.async_copy(
                    emb_hbm.at[idx_v.at[row]],
                    rows_v.at[pl.ds(j * 128, 128)],
                    sem_e,
                ))
                handles.append(pltpu.async_copy(
                    lin_hbm.at[idx_v.at[row]],
                    lin_v.at[pl.ds(j * 128, 128)],
                    sem_l,
                ))
            for h in handles:
                h.wait()

            pltpu.sync_copy(rows_v, emb_out.at[pl.ds(base, SUP_IDX)])
            pltpu.sync_copy(lin_v, lin_out.at[pl.ds(base, SUP_IDX)])
            return _

        lax.fori_loop(0, SUP, body, 0)

    return k(idx2d, emb_table, lin1d)


def _tc_body(flat_ref, lin_ref, xd_ref, wc_ref, w0d_ref, c0_ref, w1_ref,
             c1_ref, wout_ref, dw_ref, cst_ref, out_ref):
    flat = flat_ref[...]
    xd = xd_ref[...]
    g = lax.dot_general(flat, wc_ref[...], (((1,), (0,)), ((), ())),
                        preferred_element_type=jnp.float32)
    h = g[:, :H0] + lax.dot_general(xd, w0d_ref[...], (((1,), (0,)), ((), ())),
                                    preferred_element_type=jnp.float32)
    h = jnp.maximum(h + c0_ref[...], 0.0)
    h1 = lax.dot_general(h, w1_ref[...], (((1,), (0,)), ((), ())),
                         preferred_element_type=jnp.float32)
    h1 = jnp.maximum(h1 + c1_ref[...], 0.0)
    deep = jnp.sum(h1 * wout_ref[...], axis=1)

    sumvec = g[:, H0:H0 + D]
    second = 0.5 * (jnp.sum(sumvec * sumvec, axis=1)
                    - jnp.sum(flat * flat, axis=1))

    first = jnp.sum(lin_ref[...], axis=1) + jnp.sum(xd * dw_ref[...], axis=1)

    out_ref[...] = first + second + deep + cst_ref[0, 0]


def _tc_math(flat, lin_g, xd, wc, w0d, c0, w1c, c1, woutp, dwp, cst):
    grid = (B // BS,)
    return pl.pallas_call(
        _tc_body,
        grid=grid,
        in_specs=[
            pl.BlockSpec((BS, FD), lambda i: (i, 0)),
            pl.BlockSpec((BS, F), lambda i: (i, 0)),
            pl.BlockSpec((BS, 16), lambda i: (i, 0)),
            pl.BlockSpec((FD, 256), lambda i: (0, 0)),
            pl.BlockSpec((16, H0), lambda i: (0, 0)),
            pl.BlockSpec((1, H0), lambda i: (0, 0)),
            pl.BlockSpec((H0, H0), lambda i: (0, 0)),
            pl.BlockSpec((1, H0), lambda i: (0, 0)),
            pl.BlockSpec((1, H0), lambda i: (0, 0)),
            pl.BlockSpec((1, 16), lambda i: (0, 0)),
            pl.BlockSpec(memory_space=pltpu.SMEM),
        ],
        out_specs=pl.BlockSpec((BS,), lambda i: (i,)),
        out_shape=jax.ShapeDtypeStruct((B,), jnp.float32),
    )(flat, lin_g, xd, wc, w0d, c0, w1c, c1, woutp, dwp, cst)


def kernel(x_sparse, x_dense, emb_table, lin_table, lin_bias, dense_W, dense_b,
           W0, b0, g0, be0, W1, b1, g1, be1, Wout, bout, gbias):
    offsets = (jnp.arange(F, dtype=jnp.int32) * V)
    idx = x_sparse.astype(jnp.int32) + offsets[None, :]
    idx2d = idx.reshape(IDX_ROWS, 128)

    emb_lin = jax.lax.optimization_barrier(emb_table.reshape(-1)).reshape(F * V, D)
    lin1d = jax.lax.optimization_barrier(lin_table.reshape(-1))

    emb_flat, lin_flat = _sc_gather(idx2d, emb_lin, lin1d)
    flat = emb_flat.reshape(B, FD)
    lin_g = lin_flat.reshape(B, F)

    eps = 1e-5
    s0 = (g0 / jnp.sqrt(1.0 + eps))
    s1 = (g1 / jnp.sqrt(1.0 + eps))
    w0t = W0.T * s0[None, :]
    sel = jnp.zeros((FD, D), jnp.float32)
    sel = sel.at[jnp.arange(FD), jnp.arange(FD) % D].set(1.0)
    wc = jnp.zeros((FD, 256), jnp.float32)
    wc = wc.at[:, :H0].set(w0t[:FD])
    wc = wc.at[:, H0:H0 + D].set(sel)
    w0d = jnp.zeros((16, H0), jnp.float32).at[:DD].set(w0t[FD:])
    c0 = (b0 * s0 + be0).reshape(1, H0)
    w1c = jnp.zeros((H0, H0), jnp.float32).at[:, :H1].set(W1.T * s1[None, :])
    c1 = jnp.zeros((1, H0), jnp.float32).at[0, :H1].set(b1 * s1 + be1)
    woutp = jnp.zeros((1, H0), jnp.float32).at[0, :H1].set(Wout[0])
    xd = jnp.zeros((B, 16), jnp.float32).at[:, :DD].set(x_dense)
    dwp = jnp.zeros((1, 16), jnp.float32).at[0, :DD].set(dense_W[0])
    cst = (gbias[0] + lin_bias[0] + dense_b[0] + bout[0]).reshape(1, 1)

    return _tc_math(flat, lin_g, xd, wc, w0d, c0, w1c, c1, woutp, dwp, cst)
